# baseline (speedup 1.0000x reference)
import functools

import jax
import jax.numpy as jnp
from jax import lax
from jax.experimental import pallas as pl
from jax.experimental.pallas import tpu as pltpu
from jax.experimental.pallas import tpu_sc as plsc

N = 100000
E = 3200000
G = 16
NPAD = 100352
NB_N = 98
BS_N = 1024
NCORE = 2
NSUB = 16
NW = NCORE * NSUB
EPW = E // NW
CH = 2000
NCHUNK = EPW // CH
SLAB = NPAD // NSUB
B_E = 6400
NB_E = E // B_E

_mesh = plsc.VectorSubcoreMesh(core_axis_name="c", subcore_axis_name="s")


def _fill(ref, n, val, dtype):
    def body(i, _):
        ref[pl.ds(i * 16, 16)] = jnp.full((16,), val, dtype)
        return 0
    lax.fori_loop(0, n // 16, body, 0)


@functools.partial(
    pl.kernel,
    out_type=jax.ShapeDtypeStruct((NCORE, NPAD), jnp.float32),
    mesh=_mesh,
    scratch_types=[
        pltpu.VMEM((CH,), jnp.int32),
        pltpu.VMEM((CH,), jnp.float32),
        pltpu.VMEM((SLAB,), jnp.float32),
        pltpu.VMEM_SHARED((NPAD,), jnp.float32),
    ],
)
def _hist_k(row_hbm, out_hbm, idx_v, ones_v, slab_v, acc_sh):
    cid = lax.axis_index("c")
    sid = lax.axis_index("s")
    wid = sid * NCORE + cid
    _fill(ones_v, CH, 1.0, jnp.float32)
    _fill(slab_v, SLAB, 0.0, jnp.float32)
    pltpu.sync_copy(slab_v, acc_sh.at[pl.ds(sid * SLAB, SLAB)])
    plsc.subcore_barrier()

    def chunk(c, _):
        off = wid * EPW + c * CH
        pltpu.sync_copy(row_hbm.at[pl.ds(off, CH)], idx_v)
        pltpu.sync_copy(ones_v, acc_sh.at[idx_v], add=True)
        return 0

    lax.fori_loop(0, NCHUNK, chunk, 0)
    plsc.subcore_barrier()
    pltpu.sync_copy(acc_sh.at[pl.ds(sid * SLAB, SLAB)], slab_v)
    pltpu.sync_copy(slab_v, out_hbm.at[cid, pl.ds(sid * SLAB, SLAB)])


def _tables_body(cnt2_ref, b_ref, t_ref, gcn1_ref):
    j = pl.program_id(0)
    cnt_blk = cnt2_ref[...]
    csum = jnp.sum(cnt_blk, axis=0, keepdims=True)
    b_1bs = b_ref[0]
    t_ref[...] = ((b_1bs << 26) | csum.astype(jnp.int32))[None]
    iota_g = lax.broadcasted_iota(jnp.int32, (G, 1), 0)
    onehot = (b_1bs == iota_g).astype(jnp.float32)
    nz = (csum > 0.0).astype(jnp.float32)
    feats = jnp.concatenate(
        [jnp.ones((1, BS_N), jnp.float32), nz], axis=0)
    part = lax.dot_general(
        onehot, feats, (((1,), (1,)), ((), ())),
        preferred_element_type=jnp.float32)

    @pl.when(j == 0)
    def _():
        gcn1_ref[...] = part

    @pl.when(j > 0)
    def _():
        gcn1_ref[...] = gcn1_ref[...] + part


_tables = pl.pallas_call(
    _tables_body,
    grid=(NB_N,),
    in_specs=[
        pl.BlockSpec((NCORE, BS_N), lambda j: (0, j)),
        pl.BlockSpec((1, 1, BS_N), lambda j: (j, 0, 0)),
    ],
    out_specs=[
        pl.BlockSpec((1, 1, BS_N), lambda j: (j, 0, 0)),
        pl.BlockSpec((G, 2), lambda j: (0, 0)),
    ],
    out_shape=[
        jax.ShapeDtypeStruct((NB_N, 1, BS_N), jnp.int32),
        jax.ShapeDtypeStruct((G, 2), jnp.float32),
    ],
)


def _make_gather_x(eh):
    epw = eh // NW
    nch = epw // CH

    @functools.partial(
        pl.kernel,
        out_type=(
            jax.ShapeDtypeStruct((eh, 16), jnp.float32),
            jax.ShapeDtypeStruct((eh, 16), jnp.float32),
        ),
        mesh=_mesh,
        scratch_types=[
            pltpu.VMEM((CH,), jnp.int32),
            pltpu.VMEM((CH,), jnp.int32),
            pltpu.VMEM((CH, 16), jnp.float32),
            pltpu.VMEM((CH, 16), jnp.float32),
            pltpu.SemaphoreType.DMA,
            pltpu.SemaphoreType.DMA,
        ],
        compiler_params=pltpu.CompilerParams(use_tc_tiling_on_sc=False),
    )
    def gather_x(x_hbm, row_hbm, col_hbm, xr_hbm, xc_hbm,
                 ridx_v, cidx_v, xr_v, xc_v, s1, s2):
        cid = lax.axis_index("c")
        sid = lax.axis_index("s")
        wid = sid * NCORE + cid

        def chunk(c, _):
            off = wid * epw + c * CH
            pltpu.sync_copy(row_hbm.at[pl.ds(off, CH)], ridx_v)
            pltpu.sync_copy(col_hbm.at[pl.ds(off, CH)], cidx_v)
            cp1 = pltpu.async_copy(x_hbm.at[ridx_v], xr_v, s1)
            cp2 = pltpu.async_copy(x_hbm.at[cidx_v], xc_v, s2)
            cp1.wait()
            cp2.wait()
            pltpu.sync_copy(xr_v, xr_hbm.at[pl.ds(off, CH)])
            pltpu.sync_copy(xc_v, xc_hbm.at[pl.ds(off, CH)])
            return 0

        lax.fori_loop(0, nch, chunk, 0)

    return gather_x


_gather_x_h = _make_gather_x(E // 2)


def _make_gather_t(eh):
    epw = eh // NW
    nch = epw // CH

    @functools.partial(
        pl.kernel,
        out_type=jax.ShapeDtypeStruct((eh,), jnp.int32),
        mesh=_mesh,
        scratch_types=[
            pltpu.VMEM((NPAD,), jnp.int32),
            pltpu.VMEM((CH,), jnp.int32),
            pltpu.VMEM((CH,), jnp.int32),
        ],
        compiler_params=pltpu.CompilerParams(needs_layout_passes=False),
    )
    def gather_t(t_hbm, row_hbm, tout_hbm, tbl_v, ridx_v, tout_v):
        cid = lax.axis_index("c")
        sid = lax.axis_index("s")
        wid = sid * NCORE + cid
        pltpu.sync_copy(t_hbm, tbl_v)

        def chunk(c, _):
            off = wid * epw + c * CH
            pltpu.sync_copy(row_hbm.at[pl.ds(off, CH)], ridx_v)

            def g16(k, _):
                idx16 = ridx_v[pl.ds(k * 16, 16)]
                tout_v[pl.ds(k * 16, 16)] = plsc.load_gather(tbl_v, [idx16])
                return 0

            lax.fori_loop(0, CH // 16, g16, 0)
            pltpu.sync_copy(tout_v, tout_hbm.at[pl.ds(off, CH)])
            return 0

        lax.fori_loop(0, nch, chunk, 0)

    return gather_t


_gather_t_h = _make_gather_t(E // 2)


BP = B_E // 8
SUB = B_E // 8


def _edges_body(xrp_ref, xcp_ref, eap_ref, t_ref, waugt_ref, be1_ref,
                we2t_ref, be2_ref, wn1bt_ref, bn1_ref, s_ref):
    j = pl.program_id(0)
    xrt = lax.transpose(xrp_ref[...], (1, 0))
    xct = lax.transpose(xcp_ref[...], (1, 0))
    eat = lax.transpose(eap_ref[...], (1, 0))
    waugt = waugt_ref[...].astype(jnp.bfloat16)
    wct = jnp.dot(wn1bt_ref[...].astype(jnp.bfloat16),
                  we2t_ref[...].astype(jnp.bfloat16),
                  preferred_element_type=jnp.float32)
    bcc = jnp.dot(wn1bt_ref[...], be2_ref[...],
                  preferred_element_type=jnp.float32) + bn1_ref[...]
    wctb = wct.astype(jnp.bfloat16)
    t_all = t_ref[0]
    iota_g = lax.broadcasted_iota(jnp.int32, (G, 1), 0)
    part = jnp.zeros((G, 64), jnp.float32)
    for k in range(8):
        catt = jnp.concatenate(
            [lax.slice(xrt, (16 * k, 0), (16 * k + 16, BP)),
             lax.slice(xct, (16 * k, 0), (16 * k + 16, BP)),
             lax.slice(eat, (16 * k, 0), (16 * k + 16, BP))], axis=0)
        hzt = lax.dot_general(
            waugt, catt.astype(jnp.bfloat16), (((1,), (0,)), ((), ())),
            preferred_element_type=jnp.float32)
        h1t = jnp.maximum(hzt[0:64, :] + be1_ref[...], 0.0)
        zt = hzt[64:128, :]
        h2t = jnp.maximum(
            zt + jnp.dot(wctb, h1t.astype(jnp.bfloat16),
                         preferred_element_type=jnp.float32) + bcc, 0.0)
        tk = lax.slice(t_all, (0, SUB * k), (1, SUB * k + SUB))
        g = tk >> 26
        c = tk & ((1 << 26) - 1)
        w = 1.0 / jnp.maximum(c.astype(jnp.float32), 1.0)
        woh = (g == iota_g).astype(jnp.float32) * w
        part = part + lax.dot_general(
            woh, h2t, (((1,), (1,)), ((), ())),
            preferred_element_type=jnp.float32)

    @pl.when(j == 0)
    def _():
        s_ref[...] = part

    @pl.when(j > 0)
    def _():
        s_ref[...] = s_ref[...] + part


NB_H = NB_E // 2
_edges = pl.pallas_call(
    _edges_body,
    grid=(NB_H,),
    in_specs=[
        pl.BlockSpec((BP, 128), lambda j: (j, 0)),
        pl.BlockSpec((BP, 128), lambda j: (j, 0)),
        pl.BlockSpec((BP, 128), lambda j: (j, 0)),
        pl.BlockSpec((1, 1, B_E), lambda j: (j, 0, 0)),
        pl.BlockSpec((128, 48), lambda j: (0, 0)),
        pl.BlockSpec((64, 1), lambda j: (0, 0)),
        pl.BlockSpec((80, 64), lambda j: (0, 0)),
        pl.BlockSpec((80, 1), lambda j: (0, 0)),
        pl.BlockSpec((64, 80), lambda j: (0, 0)),
        pl.BlockSpec((64, 1), lambda j: (0, 0)),
    ],
    out_specs=pl.BlockSpec((G, 64), lambda j: (0, 0)),
    out_shape=jax.ShapeDtypeStruct((G, 64), jnp.float32),
    compiler_params=pltpu.CompilerParams(
        vmem_limit_bytes=50 * 1024 * 1024),
)


def _head_body(s1_ref, s2_ref, gcn1_ref, wn2_ref, bn2_ref, wfc1_ref,
               bfc1_ref, g1_ref, bt1_ref, wfc2_ref, bfc2_ref, out_ref):
    gcn1 = gcn1_ref[...]
    gc = jnp.maximum(gcn1[:, 0:1], 1.0)
    n1 = gcn1[:, 1:2]
    u2 = (jnp.dot(s1_ref[...] + s2_ref[...], wn2_ref[...],
                  preferred_element_type=jnp.float32)
          + n1 * bn2_ref[...]) / gc
    h = jnp.dot(u2, wfc1_ref[...],
                preferred_element_type=jnp.float32) + bfc1_ref[...]
    h = h * (g1_ref[...] / jnp.sqrt(1.0 + 1e-5)) + bt1_ref[...]
    h = jnp.maximum(h, 0.0)
    logits = jnp.dot(h, wfc2_ref[...],
                     preferred_element_type=jnp.float32) + bfc2_ref[...]
    mx = jnp.max(logits, axis=1, keepdims=True)
    lo = logits - mx
    out_ref[...] = lo - jnp.log(jnp.sum(jnp.exp(lo), axis=1, keepdims=True))


_head = pl.pallas_call(
    _head_body,
    out_shape=jax.ShapeDtypeStruct((G, 6), jnp.float32),
)


def kernel(x, edge_index, edge_attr, batch, W_e1, b_e1, W_e2, b_e2,
           W_n1, b_n1, W_n2, b_n2, W_fc1, b_fc1, g1, bt1, W_fc2, b_fc2):
    row = edge_index[0]
    col = edge_index[1]
    batch_pad = jnp.concatenate(
        [batch, jnp.full((NPAD - N,), 2 ** 20, jnp.int32)])
    cnt2 = _hist_k(row)
    t2, gcn1 = _tables(cnt2, batch_pad.reshape(NB_N, 1, BS_N))
    t_flat = t2.reshape(NPAD)
    w_aug = jnp.concatenate(
        [W_e1,
         jnp.concatenate([jnp.zeros((16, 64), jnp.float32), W_n1[:16],
                          jnp.zeros((16, 64), jnp.float32)], axis=0)],
        axis=1)
    EH = E // 2
    wts = (w_aug.T, b_e1.reshape(-1, 1), W_e2.T, b_e2.reshape(-1, 1),
           W_n1[16:].T, b_n1.reshape(-1, 1))
    S_halves = []
    for h in range(2):
        row_h = lax.slice(row, (h * EH,), ((h + 1) * EH,))
        col_h = lax.slice(col, (h * EH,), ((h + 1) * EH,))
        ea_h = lax.slice(edge_attr, (h * EH, 0), ((h + 1) * EH, 16))
        xr_h, xc_h = _gather_x_h(x, row_h, col_h)
        t_h = _gather_t_h(t_flat, row_h)
        tp_h = (t_h.reshape(NB_H, B_E // 8, 8)
                .transpose(0, 2, 1).reshape(NB_H, 1, B_E))
        S_halves.append(_edges(
            xr_h.reshape(EH // 8, 128), xc_h.reshape(EH // 8, 128),
            ea_h.reshape(EH // 8, 128), tp_h, *wts))
    return _head(S_halves[0], S_halves[1], gcn1, W_n2, b_n2.reshape(1, -1),
                 W_fc1, b_fc1.reshape(1, -1), g1.reshape(1, -1),
                 bt1.reshape(1, -1), W_fc2, b_fc2.reshape(1, -1))


# speedup vs baseline: 1.2906x; 1.2906x over previous
import functools

import jax
import jax.numpy as jnp
from jax import lax
from jax.experimental import pallas as pl
from jax.experimental.pallas import tpu as pltpu
from jax.experimental.pallas import tpu_sc as plsc

N = 100000
E = 3200000
G = 16
NPAD = 100352
NB_N = 98
BS_N = 1024
NCORE = 2
NSUB = 16
NW = NCORE * NSUB
EPW = E // NW
CH = 2000
NCHUNK = EPW // CH
SLAB = NPAD // NSUB
B_E = 6400
NB_E = E // B_E

_mesh = plsc.VectorSubcoreMesh(core_axis_name="c", subcore_axis_name="s")


def _fill(ref, n, val, dtype):
    def body(i, _):
        ref[pl.ds(i * 16, 16)] = jnp.full((16,), val, dtype)
        return 0
    lax.fori_loop(0, n // 16, body, 0)


@functools.partial(
    pl.kernel,
    out_type=jax.ShapeDtypeStruct((NCORE, NPAD), jnp.float32),
    mesh=_mesh,
    scratch_types=[
        pltpu.VMEM((CH,), jnp.int32),
        pltpu.VMEM((CH,), jnp.float32),
        pltpu.VMEM((SLAB,), jnp.float32),
        pltpu.VMEM_SHARED((NPAD,), jnp.float32),
    ],
)
def _hist_k(row_hbm, out_hbm, idx_v, ones_v, slab_v, acc_sh):
    cid = lax.axis_index("c")
    sid = lax.axis_index("s")
    wid = sid * NCORE + cid
    _fill(ones_v, CH, 1.0, jnp.float32)
    _fill(slab_v, SLAB, 0.0, jnp.float32)
    pltpu.sync_copy(slab_v, acc_sh.at[pl.ds(sid * SLAB, SLAB)])
    plsc.subcore_barrier()

    def chunk(c, _):
        off = wid * EPW + c * CH
        pltpu.sync_copy(row_hbm.at[pl.ds(off, CH)], idx_v)
        pltpu.sync_copy(ones_v, acc_sh.at[idx_v], add=True)
        return 0

    lax.fori_loop(0, NCHUNK, chunk, 0)
    plsc.subcore_barrier()
    pltpu.sync_copy(acc_sh.at[pl.ds(sid * SLAB, SLAB)], slab_v)
    pltpu.sync_copy(slab_v, out_hbm.at[cid, pl.ds(sid * SLAB, SLAB)])


def _tables_body(cnt2_ref, b_ref, t_ref, gcn1_ref):
    j = pl.program_id(0)
    cnt_blk = cnt2_ref[...]
    csum = jnp.sum(cnt_blk, axis=0, keepdims=True)
    b_1bs = b_ref[0]
    t_ref[...] = ((b_1bs << 26) | csum.astype(jnp.int32))[None]
    iota_g = lax.broadcasted_iota(jnp.int32, (G, 1), 0)
    onehot = (b_1bs == iota_g).astype(jnp.float32)
    nz = (csum > 0.0).astype(jnp.float32)
    feats = jnp.concatenate(
        [jnp.ones((1, BS_N), jnp.float32), nz], axis=0)
    part = lax.dot_general(
        onehot, feats, (((1,), (1,)), ((), ())),
        preferred_element_type=jnp.float32)

    @pl.when(j == 0)
    def _():
        gcn1_ref[...] = part

    @pl.when(j > 0)
    def _():
        gcn1_ref[...] = gcn1_ref[...] + part


_tables = pl.pallas_call(
    _tables_body,
    grid=(NB_N,),
    in_specs=[
        pl.BlockSpec((NCORE, BS_N), lambda j: (0, j)),
        pl.BlockSpec((1, 1, BS_N), lambda j: (j, 0, 0)),
    ],
    out_specs=[
        pl.BlockSpec((1, 1, BS_N), lambda j: (j, 0, 0)),
        pl.BlockSpec((G, 2), lambda j: (0, 0)),
    ],
    out_shape=[
        jax.ShapeDtypeStruct((NB_N, 1, BS_N), jnp.int32),
        jax.ShapeDtypeStruct((G, 2), jnp.float32),
    ],
)


@functools.partial(
    pl.kernel,
    out_type=(
        jax.ShapeDtypeStruct((E, 16), jnp.float32),
        jax.ShapeDtypeStruct((E, 16), jnp.float32),
    ),
    mesh=_mesh,
    scratch_types=[
        pltpu.VMEM((CH,), jnp.int32),
        pltpu.VMEM((CH,), jnp.int32),
        pltpu.VMEM((CH, 16), jnp.float32),
        pltpu.VMEM((CH, 16), jnp.float32),
        pltpu.SemaphoreType.DMA,
        pltpu.SemaphoreType.DMA,
    ],
    compiler_params=pltpu.CompilerParams(use_tc_tiling_on_sc=False),
)
def _gather_x(x_hbm, row_hbm, col_hbm, xr_hbm, xc_hbm,
              ridx_v, cidx_v, xr_v, xc_v, s1, s2):
    cid = lax.axis_index("c")
    sid = lax.axis_index("s")
    wid = sid * NCORE + cid

    def chunk(c, _):
        off = wid * EPW + c * CH
        pltpu.sync_copy(row_hbm.at[pl.ds(off, CH)], ridx_v)
        pltpu.sync_copy(col_hbm.at[pl.ds(off, CH)], cidx_v)
        cp1 = pltpu.async_copy(x_hbm.at[ridx_v], xr_v, s1)
        cp2 = pltpu.async_copy(x_hbm.at[cidx_v], xc_v, s2)
        cp1.wait()
        cp2.wait()
        pltpu.sync_copy(xr_v, xr_hbm.at[pl.ds(off, CH)])
        pltpu.sync_copy(xc_v, xc_hbm.at[pl.ds(off, CH)])
        return 0

    lax.fori_loop(0, NCHUNK, chunk, 0)


@functools.partial(
    pl.kernel,
    out_type=jax.ShapeDtypeStruct((E,), jnp.int32),
    mesh=_mesh,
    scratch_types=[
        pltpu.VMEM((NPAD,), jnp.int32),
        pltpu.VMEM((CH,), jnp.int32),
        pltpu.VMEM((CH,), jnp.int32),
    ],
    compiler_params=pltpu.CompilerParams(needs_layout_passes=False),
)
def _gather_t(t_hbm, row_hbm, tout_hbm, tbl_v, ridx_v, tout_v):
    cid = lax.axis_index("c")
    sid = lax.axis_index("s")
    wid = sid * NCORE + cid
    pltpu.sync_copy(t_hbm, tbl_v)

    def chunk(c, _):
        off = wid * EPW + c * CH
        pltpu.sync_copy(row_hbm.at[pl.ds(off, CH)], ridx_v)

        def g16(k, _):
            idx16 = ridx_v[pl.ds(k * 16, 16)]
            tout_v[pl.ds(k * 16, 16)] = plsc.load_gather(tbl_v, [idx16])
            return 0

        lax.fori_loop(0, CH // 16, g16, 0)
        pltpu.sync_copy(tout_v, tout_hbm.at[pl.ds(off, CH)])
        return 0

    lax.fori_loop(0, NCHUNK, chunk, 0)


BP = B_E // 8
SUB = B_E // 8


def _edges_body(xrp_ref, xcp_ref, eap_ref, t_ref, waugt_ref, be1_ref,
                we2t_ref, be2_ref, wn1bt_ref, bn1_ref, s_ref):
    j = pl.program_id(0)
    xrt = lax.transpose(xrp_ref[...], (1, 0))
    xct = lax.transpose(xcp_ref[...], (1, 0))
    eat = lax.transpose(eap_ref[...], (1, 0))
    waugt = waugt_ref[...].astype(jnp.bfloat16)
    wct = jnp.dot(wn1bt_ref[...].astype(jnp.bfloat16),
                  we2t_ref[...].astype(jnp.bfloat16),
                  preferred_element_type=jnp.float32)
    bcc = jnp.dot(wn1bt_ref[...], be2_ref[...],
                  preferred_element_type=jnp.float32) + bn1_ref[...]
    wctb = wct.astype(jnp.bfloat16)
    t_all = t_ref[0]
    iota_g = lax.broadcasted_iota(jnp.int32, (G, 1), 0)
    part = jnp.zeros((G, 64), jnp.float32)
    for k in range(8):
        catt = jnp.concatenate(
            [lax.slice(xrt, (16 * k, 0), (16 * k + 16, BP)),
             lax.slice(xct, (16 * k, 0), (16 * k + 16, BP)),
             lax.slice(eat, (16 * k, 0), (16 * k + 16, BP))], axis=0)
        hzt = lax.dot_general(
            waugt, catt.astype(jnp.bfloat16), (((1,), (0,)), ((), ())),
            preferred_element_type=jnp.float32)
        h1t = jnp.maximum(hzt[0:64, :] + be1_ref[...], 0.0)
        zt = hzt[64:128, :]
        h2t = jnp.maximum(
            zt + jnp.dot(wctb, h1t.astype(jnp.bfloat16),
                         preferred_element_type=jnp.float32) + bcc, 0.0)
        tk = lax.slice(t_all, (0, SUB * k), (1, SUB * k + SUB))
        g = tk >> 26
        c = tk & ((1 << 26) - 1)
        w = 1.0 / jnp.maximum(c.astype(jnp.float32), 1.0)
        woh = (g == iota_g).astype(jnp.float32) * w
        part = part + lax.dot_general(
            woh, h2t, (((1,), (1,)), ((), ())),
            preferred_element_type=jnp.float32)

    @pl.when(j == 0)
    def _():
        s_ref[...] = part

    @pl.when(j > 0)
    def _():
        s_ref[...] = s_ref[...] + part


_edges = pl.pallas_call(
    _edges_body,
    grid=(NB_E,),
    in_specs=[
        pl.BlockSpec((BP, 128), lambda j: (j, 0)),
        pl.BlockSpec((BP, 128), lambda j: (j, 0)),
        pl.BlockSpec((BP, 128), lambda j: (j, 0)),
        pl.BlockSpec((1, 1, B_E), lambda j: (j, 0, 0)),
        pl.BlockSpec((128, 48), lambda j: (0, 0)),
        pl.BlockSpec((64, 1), lambda j: (0, 0)),
        pl.BlockSpec((80, 64), lambda j: (0, 0)),
        pl.BlockSpec((80, 1), lambda j: (0, 0)),
        pl.BlockSpec((64, 80), lambda j: (0, 0)),
        pl.BlockSpec((64, 1), lambda j: (0, 0)),
    ],
    out_specs=pl.BlockSpec((G, 64), lambda j: (0, 0)),
    out_shape=jax.ShapeDtypeStruct((G, 64), jnp.float32),
    compiler_params=pltpu.CompilerParams(
        vmem_limit_bytes=50 * 1024 * 1024),
)


def _head_body(s_ref, gcn1_ref, wn2_ref, bn2_ref, wfc1_ref, bfc1_ref,
               g1_ref, bt1_ref, wfc2_ref, bfc2_ref, out_ref):
    gcn1 = gcn1_ref[...]
    gc = jnp.maximum(gcn1[:, 0:1], 1.0)
    n1 = gcn1[:, 1:2]
    u2 = (jnp.dot(s_ref[...], wn2_ref[...],
                  preferred_element_type=jnp.float32)
          + n1 * bn2_ref[...]) / gc
    h = jnp.dot(u2, wfc1_ref[...],
                preferred_element_type=jnp.float32) + bfc1_ref[...]
    h = h * (g1_ref[...] / jnp.sqrt(1.0 + 1e-5)) + bt1_ref[...]
    h = jnp.maximum(h, 0.0)
    logits = jnp.dot(h, wfc2_ref[...],
                     preferred_element_type=jnp.float32) + bfc2_ref[...]
    mx = jnp.max(logits, axis=1, keepdims=True)
    lo = logits - mx
    out_ref[...] = lo - jnp.log(jnp.sum(jnp.exp(lo), axis=1, keepdims=True))


_head = pl.pallas_call(
    _head_body,
    out_shape=jax.ShapeDtypeStruct((G, 6), jnp.float32),
)


def kernel(x, edge_index, edge_attr, batch, W_e1, b_e1, W_e2, b_e2,
           W_n1, b_n1, W_n2, b_n2, W_fc1, b_fc1, g1, bt1, W_fc2, b_fc2):
    row = edge_index[0]
    col = edge_index[1]
    batch_pad = jnp.concatenate(
        [batch, jnp.full((NPAD - N,), 2 ** 20, jnp.int32)])
    cnt2 = _hist_k(row)
    t2, gcn1 = _tables(cnt2, batch_pad.reshape(NB_N, 1, BS_N))
    xr, xc = _gather_x(x, row, col)
    t_e = _gather_t(t2.reshape(NPAD), row)
    t_perm = (t_e.reshape(NB_E, B_E // 8, 8)
              .transpose(0, 2, 1).reshape(NB_E, 1, B_E))
    w_aug = jnp.concatenate(
        [W_e1,
         jnp.concatenate([jnp.zeros((16, 64), jnp.float32), W_n1[:16],
                          jnp.zeros((16, 64), jnp.float32)], axis=0)],
        axis=1)
    S = _edges(xr.reshape(E // 8, 128), xc.reshape(E // 8, 128),
               edge_attr.reshape(E // 8, 128), t_perm,
               w_aug.T, b_e1.reshape(-1, 1), W_e2.T, b_e2.reshape(-1, 1),
               W_n1[16:].T, b_n1.reshape(-1, 1))
    return _head(S, gcn1, W_n2, b_n2.reshape(1, -1),
                 W_fc1, b_fc1.reshape(1, -1), g1.reshape(1, -1),
                 bt1.reshape(1, -1), W_fc2, b_fc2.reshape(1, -1))
